# baseline (speedup 1.0000x reference)
import jax
import jax.numpy as jnp
from jax import lax
from jax.experimental import pallas as pl
from jax.experimental.pallas import tpu as pltpu
from jax.experimental.pallas import tpu_sc as plsc

B = 8
N_FINE = 40962
D = 128
K = 7
N_COARSE = 10242

NW = 32
PER_W = 320
N_MAIN = NW * PER_W
N_TAIL = N_COARSE - N_MAIN
CHUNK = 16
NCHUNK = PER_W // CHUNK
IDX_C = CHUNK * K
DBLK = D // 16


def _tree_max7(rows):
    t0 = jnp.maximum(rows[0], rows[1])
    t1 = jnp.maximum(rows[2], rows[3])
    t2 = jnp.maximum(rows[4], rows[5])
    return jnp.maximum(jnp.maximum(t0, t1), jnp.maximum(t2, rows[6]))


NBUF = 4


def _pool_body(x_hbm, idxm_hbm, idxt_hbm, out_hbm,
               idx_v, gbuf0, gbuf1, gbuf2, gbuf3, obuf,
               idxt_raw, gtail, otail,
               sem0, sem1, sem2, sem3, semS, semT):
    gbufs = (gbuf0, gbuf1, gbuf2, gbuf3)
    sems = (sem0, sem1, sem2, sem3)
    wid = lax.axis_index("s") * 2 + lax.axis_index("c")
    base_c = wid * PER_W
    pltpu.sync_copy(idxm_hbm.at[pl.ds(wid * (PER_W * K), PER_W * K)], idx_v)

    def fire(gbuf, sem, j, b):
        idx_slice = idx_v.at[pl.ds(j * IDX_C, IDX_C)]
        pltpu.async_copy(x_hbm.at[b].at[idx_slice], gbuf, sem)

    def wait_gather(gbuf, sem):
        idx_slice = idx_v.at[pl.ds(0, IDX_C)]
        pltpu.make_async_copy(x_hbm.at[0].at[idx_slice], gbuf, sem).wait()

    def wait_store():
        pltpu.make_async_copy(obuf, out_hbm.at[0, pl.ds(0, PER_W)],
                              semS).wait()

    def compute(gbuf, row_base):
        @plsc.parallel_loop(0, CHUNK, 1, unroll=2)
        def _(c):
            for dblk in range(DBLK):
                o = pl.ds(dblk * 16, 16)
                m = _tree_max7([gbuf[K * c + k2, o] for k2 in range(K)])
                obuf[row_base + c, o] = m

    for p in range(NBUF):
        fire(gbufs[p], sems[p], p, 0)

    def batch_body(b, carry):
        @pl.when(b >= 1)
        def _():
            wait_store()

        def quad_body(j4, carry2):
            c0 = NBUF * j4
            for p in range(NBUF):
                wait_gather(gbufs[p], sems[p])
                compute(gbufs[p], (c0 + p) * CHUNK)
                fire(gbufs[p], sems[p], c0 + p + NBUF, b)
            return carry2

        lax.fori_loop(0, NCHUNK // NBUF - 1, quad_body, 0)

        for p in range(NBUF):
            wait_gather(gbufs[p], sems[p])
            compute(gbufs[p], (NCHUNK - NBUF + p) * CHUNK)

            @pl.when(b < B - 1)
            def _():
                fire(gbufs[p], sems[p], p, b + 1)

        pltpu.async_copy(obuf, out_hbm.at[b, pl.ds(base_c, PER_W)], semS)
        return carry

    lax.fori_loop(0, B, batch_body, 0)
    wait_store()

    @pl.when(wid == NW - 1)
    def _():
        pltpu.sync_copy(idxt_hbm, idxt_raw)

        def tfire(b, carry):
            pltpu.async_copy(x_hbm.at[b].at[idxt_raw], gtail.at[b], semT)
            return carry

        lax.fori_loop(0, B, tfire, 0)

        def tdrain(b, carry):
            pltpu.make_async_copy(x_hbm.at[b].at[idxt_raw], gtail.at[b],
                                  semT).wait()
            for c in range(N_TAIL):
                for dblk in range(DBLK):
                    o = pl.ds(dblk * 16, 16)
                    m = _tree_max7([gtail[b, K * c + k2, o]
                                    for k2 in range(K)])
                    otail[c, o] = m
            pltpu.sync_copy(otail, out_hbm.at[b, pl.ds(N_MAIN, N_TAIL)])
            return carry

        lax.fori_loop(0, B, tdrain, 0)


def kernel(x, pool_idx):
    idx = pool_idx.astype(jnp.int32)
    idx_main = idx[:N_MAIN].reshape(N_MAIN * K)
    idx_tail = jnp.pad(idx[N_MAIN:].reshape(N_TAIL * K), (0, 16 - N_TAIL * K))

    mesh = plsc.VectorSubcoreMesh(core_axis_name="c", subcore_axis_name="s")
    f = pl.kernel(
        _pool_body,
        mesh=mesh,
        compiler_params=pltpu.CompilerParams(use_tc_tiling_on_sc=True),
        out_type=jax.ShapeDtypeStruct((B, N_COARSE, D), jnp.float32),
        scratch_types=[
            pltpu.VMEM((PER_W * K,), jnp.int32),
            pltpu.VMEM((IDX_C, D), jnp.float32),
            pltpu.VMEM((IDX_C, D), jnp.float32),
            pltpu.VMEM((IDX_C, D), jnp.float32),
            pltpu.VMEM((IDX_C, D), jnp.float32),
            pltpu.VMEM((PER_W, D), jnp.float32),
            pltpu.VMEM((16,), jnp.int32),
            pltpu.VMEM((B, 16, D), jnp.float32),
            pltpu.VMEM((N_TAIL, D), jnp.float32),
            pltpu.SemaphoreType.DMA,
            pltpu.SemaphoreType.DMA,
            pltpu.SemaphoreType.DMA,
            pltpu.SemaphoreType.DMA,
            pltpu.SemaphoreType.DMA,
            pltpu.SemaphoreType.DMA,
        ],
    )
    return f(x, idx_main, idx_tail)


# speedup vs baseline: 2.0105x; 2.0105x over previous
import jax
import jax.numpy as jnp
from jax import lax
from jax.experimental import pallas as pl
from jax.experimental.pallas import tpu as pltpu
from jax.experimental.pallas import tpu_sc as plsc

B = 8
N_FINE = 40962
D = 128
K = 7
N_COARSE = 10242

NW = 32
PER_W = 320
N_MAIN = NW * PER_W
N_TAIL = N_COARSE - N_MAIN
CHUNK = 2
NCHUNK = PER_W // CHUNK
IDX_C = CHUNK * K
IDX_P = 16
NBUF = 4
DBLK = D // 16


def _tree_max7(rows):
    t0 = jnp.maximum(rows[0], rows[1])
    t1 = jnp.maximum(rows[2], rows[3])
    t2 = jnp.maximum(rows[4], rows[5])
    return jnp.maximum(jnp.maximum(t0, t1), jnp.maximum(t2, rows[6]))


def _pool_body(x_hbm, idxm_hbm, idxt_hbm, out_hbm,
               idx_v, gbuf0, gbuf1, gbuf2, gbuf3,
               obuf0, obuf1, obuf2, obuf3, idxt_raw,
               semG0, semG1, semG2, semG3, semS, semT):
    gbufs = (gbuf0, gbuf1, gbuf2, gbuf3)
    obufs = (obuf0, obuf1, obuf2, obuf3)
    semGs = (semG0, semG1, semG2, semG3)
    wid = lax.axis_index("s") * 2 + lax.axis_index("c")
    base_c = wid * PER_W
    pltpu.sync_copy(idxm_hbm.at[pl.ds(wid * (NCHUNK * IDX_P), NCHUNK * IDX_P)],
                    idx_v)

    def fire(p, j):
        idx_slice = idx_v.at[pl.ds(j * IDX_P, IDX_C)]
        pltpu.async_copy(x_hbm.at[idx_slice], gbufs[p], semGs[p])

    def wait_gather(p):
        idx_slice = idx_v.at[pl.ds(0, IDX_C)]
        pltpu.make_async_copy(x_hbm.at[idx_slice], gbufs[p], semGs[p]).wait()

    def wait_store(p):
        pltpu.make_async_copy(obufs[p], out_hbm.at[pl.ds(0, CHUNK)],
                              semS).wait()

    def compute(p):
        gbuf = gbufs[p]
        obuf = obufs[p]

        @plsc.parallel_loop(0, CHUNK * B, 1, unroll=2)
        def _(i):
            c = lax.shift_right_logical(i, 3)
            b = lax.bitwise_and(i, 7)
            for dblk in range(DBLK):
                o = pl.ds(dblk * 16, 16)
                m = _tree_max7([gbuf[K * c + k2, b, o] for k2 in range(K)])
                obuf[c, b, o] = m

    for p in range(NBUF):
        fire(p, p)

    def quad_body(j4, carry):
        @pl.when(j4 >= 1)
        def _():
            for p in range(NBUF):
                wait_store(p)

        for p in range(NBUF):
            c = NBUF * j4 + p
            wait_gather(p)
            compute(p)
            pltpu.async_copy(obufs[p],
                             out_hbm.at[pl.ds(base_c + c * CHUNK, CHUNK)],
                             semS)

            @pl.when(c + NBUF < NCHUNK)
            def _():
                fire(p, c + NBUF)

        return carry

    lax.fori_loop(0, NCHUNK // NBUF, quad_body, 0)
    for p in range(NBUF):
        wait_store(p)

    @pl.when(wid == NW - 1)
    def _():
        pltpu.sync_copy(idxt_hbm, idxt_raw)
        idx_slice = idxt_raw.at[pl.ds(0, IDX_C)]
        pltpu.async_copy(x_hbm.at[idx_slice], gbuf0, semT).wait()
        for c in range(N_TAIL):
            for b in range(B):
                for dblk in range(DBLK):
                    o = pl.ds(dblk * 16, 16)
                    m = _tree_max7([gbuf0[K * c + k2, b, o]
                                    for k2 in range(K)])
                    obuf0[c, b, o] = m
        pltpu.sync_copy(obuf0, out_hbm.at[pl.ds(N_MAIN, N_TAIL)])


def kernel(x, pool_idx):
    idx = pool_idx.astype(jnp.int32)
    idx_main = jnp.pad(idx[:N_MAIN].reshape(NW * NCHUNK, IDX_C),
                       ((0, 0), (0, IDX_P - IDX_C))).reshape(-1)
    idx_tail = jnp.pad(idx[N_MAIN:].reshape(N_TAIL * K), (0, 16 - N_TAIL * K))
    x_t = jnp.transpose(x, (1, 0, 2))

    mesh = plsc.VectorSubcoreMesh(core_axis_name="c", subcore_axis_name="s")
    f = pl.kernel(
        _pool_body,
        mesh=mesh,
        out_type=jax.ShapeDtypeStruct((N_COARSE, B, D), jnp.float32),
        scratch_types=[
            pltpu.VMEM((NCHUNK * IDX_P,), jnp.int32),
            pltpu.VMEM((IDX_C, B, D), jnp.float32),
            pltpu.VMEM((IDX_C, B, D), jnp.float32),
            pltpu.VMEM((IDX_C, B, D), jnp.float32),
            pltpu.VMEM((IDX_C, B, D), jnp.float32),
            pltpu.VMEM((CHUNK, B, D), jnp.float32),
            pltpu.VMEM((CHUNK, B, D), jnp.float32),
            pltpu.VMEM((CHUNK, B, D), jnp.float32),
            pltpu.VMEM((CHUNK, B, D), jnp.float32),
            pltpu.VMEM((16,), jnp.int32),
            pltpu.SemaphoreType.DMA,
            pltpu.SemaphoreType.DMA,
            pltpu.SemaphoreType.DMA,
            pltpu.SemaphoreType.DMA,
            pltpu.SemaphoreType.DMA,
            pltpu.SemaphoreType.DMA,
        ],
    )
    out_t = f(x_t, idx_main, idx_tail)
    return jnp.transpose(out_t, (1, 0, 2))
